# baseline (speedup 1.0000x reference)
import functools

import jax
import jax.numpy as jnp
from jax import lax
from jax.experimental import pallas as pl
from jax.experimental.pallas import tpu as pltpu
from jax.experimental.pallas import tpu_sc as plsc

F32 = jnp.float32

B = 1024
N_TILE = 225
N_ENT = 100
VOCAB = 4096
EMBED = 64

NC = 2
NS = 16
NW = NC * NS
L = 16

P_TILE = B * N_TILE
P_ENT = B * N_ENT
PW_TILE = P_TILE // NW
PW_ENT = P_ENT // NW
ROWS_PER_W = 32


def _tc_pre_body(tt, te, wft, wfe, wat, wae, wcbt, wcbe,
                 t2t, t2e, sdt, sde, mt, me):
    hi = lax.Precision.HIGHEST
    z = jnp.zeros((EMBED, 128 - EMBED - 1), F32)
    t2t[...] = jnp.dot(tt[...], wft[...], precision=hi)
    t2e[...] = jnp.dot(te[...], wfe[...], precision=hi)
    sdt[...] = jnp.dot(tt[...], wat[...], precision=hi)
    sde[...] = jnp.dot(te[...], wae[...], precision=hi)
    wt = jnp.concatenate([wft[...], wat[...], z], axis=1)
    we = jnp.concatenate([wfe[...], wae[...], z], axis=1)
    mt[...] = jnp.dot(wcbt[...], wt, precision=hi)
    me[...] = jnp.dot(wcbe[...], we, precision=hi)


def _tc_pre(tt, te, wft, wfe, wat, wae, wcbt, wcbe):
    out_shape = (
        jax.ShapeDtypeStruct((VOCAB, EMBED), F32),
        jax.ShapeDtypeStruct((VOCAB, EMBED), F32),
        jax.ShapeDtypeStruct((VOCAB, 1), F32),
        jax.ShapeDtypeStruct((VOCAB, 1), F32),
        jax.ShapeDtypeStruct((2, 128), F32),
        jax.ShapeDtypeStruct((2, 128), F32),
    )
    return pl.pallas_call(_tc_pre_body, out_shape=out_shape)(
        tt, te, wft, wfe, wat, wae, wcbt, wcbe)


def _sc_fused(cont_t, disc_t, cont_e, disc_e,
              sd_t, t2_t, par_t, sd_e, t2_e, par_e):
    mesh = plsc.VectorSubcoreMesh(core_axis_name="c", subcore_axis_name="s")

    @functools.partial(
        pl.kernel, mesh=mesh,
        out_type=(jax.ShapeDtypeStruct((P_TILE, EMBED), F32),
                  jax.ShapeDtypeStruct((P_ENT, EMBED), F32)),
        compiler_params=pltpu.CompilerParams(
            needs_layout_passes=False, use_tc_tiling_on_sc=False),
        scratch_types=(
            [pltpu.VMEM((VOCAB,), F32)]
            + [pltpu.VMEM((256,), F32)]
            + [pltpu.VMEM((PW_TILE * 4,), F32)]
            + [pltpu.VMEM((PW_TILE * 3,), jnp.int32)]
            + [pltpu.VMEM((240,), jnp.int32)] * 3
            + [pltpu.VMEM((N_TILE, EMBED), F32)] * 3
            + [pltpu.VMEM((N_ENT, EMBED), F32)] * 3
            + [pltpu.VMEM((240,), F32)] * 5
            + [pltpu.SemaphoreType.DMA]
        ),
    )
    def k(cont_t_hbm, disc_t_hbm, cont_e_hbm, disc_e_hbm,
          sd_t_hbm, t2_t_hbm, par_t_hbm, sd_e_hbm, t2_e_hbm, par_e_hbm,
          out_t_hbm, out_e_hbm,
          sd_v, par_v, cont_v, disc_v, d0c, d1c, d2c,
          r0t, r1t, r2t, r0e, r1e, r2e,
          pv0, pv1, pv2, pvw, pvs, sem):
        wid = lax.axis_index("s") * NC + lax.axis_index("c")
        iota = lax.iota(jnp.int32, L)
        i3 = [iota * 3 + a for a in range(3)]
        i4 = [iota * 4 + a for a in range(4)]
        dca = (d0c, d1c, d2c)

        def phase(n, per_w, cont_hbm, disc_hbm, sd_hbm, par_hbm, t2_hbm,
                  out_hbm, r0, r1, r2):
            n_grp = -(-n // L)
            tail = n - (n_grp - 1) * L
            base_w = wid * per_w
            pltpu.sync_copy(sd_hbm, sd_v)
            pltpu.sync_copy(par_hbm, par_v)
            cdst = cont_v if per_w == PW_TILE else cont_v.at[pl.ds(0, per_w * 4)]
            ddst = disc_v if per_w == PW_TILE else disc_v.at[pl.ds(0, per_w * 3)]
            pltpu.sync_copy(cont_hbm.at[pl.ds(base_w * 4, per_w * 4)], cdst)
            pltpu.sync_copy(disc_hbm.at[pl.ds(base_w * 3, per_w * 3)], ddst)
            uvec = [par_v[pl.ds(16 * q, L)] for q in range(4)]
            vvec = [par_v[pl.ds(128 + 16 * q, L)] for q in range(4)]
            alpha = par_v[pl.ds(64, L)][0]
            beta = par_v[pl.ds(192, L)][0]

            def deint_grp(gb, b3, clamp):
                for a in range(3):
                    idx = b3 + gb * 3 + i3[a]
                    if clamp:
                        idx = jnp.minimum(idx, per_w * 3 - 1)
                    dca[a][pl.ds(gb, L)] = plsc.load_gather(disc_v, [idx])

            def soft_grp(gb, b4, cent2, cent3, clamp):
                cs = []
                for a in range(4):
                    idx = b4 + gb * 4 + i4[a]
                    if clamp:
                        idx = jnp.minimum(idx, per_w * 4 - 1)
                    cs.append(plsc.load_gather(cont_v, [idx]))
                c0, c1, c2v, c3v = cs
                cc2 = cent2 - c2v
                cc3 = cent3 - c3v
                s0 = alpha * c0 + beta
                s1 = alpha * c1 + beta
                s2 = alpha * cc2 + beta
                s3 = alpha * cc3 + beta
                s4 = plsc.load_gather(sd_v, [d0c[pl.ds(gb, L)]])
                s5 = plsc.load_gather(sd_v, [d1c[pl.ds(gb, L)]])
                s6 = plsc.load_gather(sd_v, [d2c[pl.ds(gb, L)]])
                m = jnp.maximum(
                    jnp.maximum(jnp.maximum(s0, s1), jnp.maximum(s2, s3)),
                    jnp.maximum(jnp.maximum(s4, s5), s6))
                e0 = jnp.exp(s0 - m)
                e1 = jnp.exp(s1 - m)
                e2 = jnp.exp(s2 - m)
                e3 = jnp.exp(s3 - m)
                e4 = jnp.exp(s4 - m)
                e5 = jnp.exp(s5 - m)
                e6 = jnp.exp(s6 - m)
                rr = 1.0 / (((e0 + e1) + (e2 + e3)) + ((e4 + e5) + e6))
                p0 = e0 * rr
                p1 = e1 * rr
                p2 = e2 * rr
                p3 = e3 * rr
                pv0[pl.ds(gb, L)] = e4 * rr
                pv1[pl.ds(gb, L)] = e5 * rr
                pv2[pl.ds(gb, L)] = e6 * rr
                pvw[pl.ds(gb, L)] = p0 * c0 + p1 * c1 + p2 * cc2 + p3 * cc3
                pvs[pl.ds(gb, L)] = (p0 + p1) + (p2 + p3)

            def comb_grp(gb, jn):
                p4g = pv0[pl.ds(gb, L)]
                p5g = pv1[pl.ds(gb, L)]
                p6g = pv2[pl.ds(gb, L)]
                wg = pvw[pl.ds(gb, L)]
                psg = pvs[pl.ds(gb, L)]
                for j in range(jn):
                    i = gb + j
                    p4 = p4g[j]
                    p5 = p5g[j]
                    p6 = p6g[j]
                    w = wg[j]
                    ps = psg[j]
                    for q in range(4):
                        sl = pl.ds(q * L, L)
                        r1[i, sl] = (r0[i, sl] * p4 + r1[i, sl] * p5
                                     + r2[i, sl] * p6
                                     + uvec[q] * w + vvec[q] * ps)

            def chunk(ci, carry):
                o = ci * n
                b3 = o * 3
                b4 = o * 4
                cv = cont_v[pl.ds(b4, L)]
                cent2 = cv[2]
                cent3 = cv[3]

                def dg(g, c2):
                    deint_grp(g * L, b3, False)
                    return c2
                lax.fori_loop(0, n_grp - 1, dg, 0)
                deint_grp((n_grp - 1) * L, b3, True)

                cps = [pltpu.async_copy(
                    t2_hbm.at[dca[a].at[pl.ds(0, n)]], (r0, r1, r2)[a], sem)
                    for a in range(3)]

                def sg(g, c2):
                    soft_grp(g * L, b4, cent2, cent3, False)
                    return c2
                lax.fori_loop(0, n_grp - 1, sg, 0)
                soft_grp((n_grp - 1) * L, b4, cent2, cent3, True)

                for cp in cps:
                    cp.wait()

                def cg(g, c2):
                    comb_grp(g * L, L)
                    return c2
                lax.fori_loop(0, n_grp - 1, cg, 0)
                comb_grp((n_grp - 1) * L, tail)

                pltpu.sync_copy(r1, out_hbm.at[pl.ds(base_w + o, n)])
                return carry

            lax.fori_loop(0, ROWS_PER_W, chunk, 0)

        phase(N_TILE, PW_TILE, cont_t_hbm, disc_t_hbm, sd_t_hbm, par_t_hbm,
              t2_t_hbm, out_t_hbm, r0t, r1t, r2t)
        phase(N_ENT, PW_ENT, cont_e_hbm, disc_e_hbm, sd_e_hbm, par_e_hbm,
              t2_e_hbm, out_e_hbm, r0e, r1e, r2e)

    return k(cont_t, disc_t, cont_e, disc_e,
             sd_t, t2_t, par_t, sd_e, t2_e, par_e)


def kernel(tile_cont, tile_disc, ent_cont, ent_disc, tile_table, ent_table,
           tile_wc, tile_bc, tile_wa, tile_wf,
           ent_wc, ent_bc, ent_wa, ent_wf):
    wcbt = jnp.stack([tile_wc, tile_bc])
    wcbe = jnp.stack([ent_wc, ent_bc])

    t2t, t2e, sdt, sde, mt, me = _tc_pre(
        tile_table, ent_table, tile_wf, ent_wf,
        tile_wa[:, None], ent_wa[:, None], wcbt, wcbe)

    out_t, out_e = _sc_fused(
        tile_cont.reshape(P_TILE * 4),
        tile_disc.astype(jnp.int32).reshape(P_TILE * 3),
        ent_cont.reshape(P_ENT * 4),
        ent_disc.astype(jnp.int32).reshape(P_ENT * 3),
        sdt.reshape(VOCAB), t2t, mt.reshape(256),
        sde.reshape(VOCAB), t2e, me.reshape(256))

    return (out_t.reshape(B, N_TILE, EMBED), out_e.reshape(B, N_ENT, EMBED))


# speedup vs baseline: 1.0045x; 1.0045x over previous
import functools

import jax
import jax.numpy as jnp
from jax import lax
from jax.experimental import pallas as pl
from jax.experimental.pallas import tpu as pltpu
from jax.experimental.pallas import tpu_sc as plsc

F32 = jnp.float32

B = 1024
N_TILE = 225
N_ENT = 100
VOCAB = 4096
EMBED = 64

NC = 2
NS = 16
NW = NC * NS
L = 16

P_TILE = B * N_TILE
P_ENT = B * N_ENT
PW_TILE = P_TILE // NW
PW_ENT = P_ENT // NW
ROWS_PER_W = 32


def _tc_pre_body(tt, te, wet, wee, wcbt, wcbe, t2t, t2e, mt, me):
    hi = lax.Precision.HIGHEST
    t2t[...] = jnp.dot(tt[...], wet[...], precision=hi)
    t2e[...] = jnp.dot(te[...], wee[...], precision=hi)
    mt[...] = jnp.dot(wcbt[...], wet[...], precision=hi)
    me[...] = jnp.dot(wcbe[...], wee[...], precision=hi)


def _tc_pre(tt, te, wet, wee, wcbt, wcbe):
    out_shape = (
        jax.ShapeDtypeStruct((VOCAB, 128), F32),
        jax.ShapeDtypeStruct((VOCAB, 128), F32),
        jax.ShapeDtypeStruct((2, 128), F32),
        jax.ShapeDtypeStruct((2, 128), F32),
    )
    return pl.pallas_call(_tc_pre_body, out_shape=out_shape)(
        tt, te, wet, wee, wcbt, wcbe)


def _sc_fused(cont_t, disc_t, cont_e, disc_e,
              sd_t, t2_t, par_t, sd_e, t2_e, par_e):
    mesh = plsc.VectorSubcoreMesh(core_axis_name="c", subcore_axis_name="s")

    @functools.partial(
        pl.kernel, mesh=mesh,
        out_type=(jax.ShapeDtypeStruct((P_TILE, EMBED), F32),
                  jax.ShapeDtypeStruct((P_ENT, EMBED), F32)),
        compiler_params=pltpu.CompilerParams(
            needs_layout_passes=False, use_tc_tiling_on_sc=False),
        scratch_types=(
            [pltpu.VMEM((VOCAB,), F32)]
            + [pltpu.VMEM((256,), F32)]
            + [pltpu.VMEM((PW_TILE * 4,), F32)]
            + [pltpu.VMEM((PW_TILE * 3,), jnp.int32)]
            + [pltpu.VMEM((240,), jnp.int32)] * 3
            + [pltpu.VMEM((N_TILE, EMBED), F32)] * 3
            + [pltpu.VMEM((N_ENT, EMBED), F32)] * 3
            + [pltpu.VMEM((240,), F32)] * 5
            + [pltpu.SemaphoreType.DMA]
        ),
    )
    def k(cont_t_hbm, disc_t_hbm, cont_e_hbm, disc_e_hbm,
          sd_t_hbm, t2_t_hbm, par_t_hbm, sd_e_hbm, t2_e_hbm, par_e_hbm,
          out_t_hbm, out_e_hbm,
          sd_v, par_v, cont_v, disc_v, d0c, d1c, d2c,
          r0t, r1t, r2t, r0e, r1e, r2e,
          pv0, pv1, pv2, pvw, pvs, sem):
        wid = lax.axis_index("s") * NC + lax.axis_index("c")
        iota = lax.iota(jnp.int32, L)
        i3 = [iota * 3 + a for a in range(3)]
        i4 = [iota * 4 + a for a in range(4)]
        dca = (d0c, d1c, d2c)

        def phase(n, per_w, cont_hbm, disc_hbm, sd_hbm, par_hbm, t2_hbm,
                  out_hbm, r0, r1, r2):
            n_grp = -(-n // L)
            tail = n - (n_grp - 1) * L
            base_w = wid * per_w
            pltpu.sync_copy(sd_hbm, sd_v)
            pltpu.sync_copy(par_hbm, par_v)
            cdst = cont_v if per_w == PW_TILE else cont_v.at[pl.ds(0, per_w * 4)]
            ddst = disc_v if per_w == PW_TILE else disc_v.at[pl.ds(0, per_w * 3)]
            pltpu.sync_copy(cont_hbm.at[pl.ds(base_w * 4, per_w * 4)], cdst)
            pltpu.sync_copy(disc_hbm.at[pl.ds(base_w * 3, per_w * 3)], ddst)
            uvec = [par_v[pl.ds(16 * q, L)] for q in range(4)]
            vvec = [par_v[pl.ds(128 + 16 * q, L)] for q in range(4)]
            alpha = par_v[pl.ds(64, L)][0]
            beta = par_v[pl.ds(192, L)][0]

            def deint_grp(gb, b3, clamp):
                for a in range(3):
                    idx = b3 + gb * 3 + i3[a]
                    if clamp:
                        idx = jnp.minimum(idx, per_w * 3 - 1)
                    dca[a][pl.ds(gb, L)] = plsc.load_gather(disc_v, [idx])

            def soft_grp(gb, b4, cent2, cent3, clamp):
                cs = []
                for a in range(4):
                    idx = b4 + gb * 4 + i4[a]
                    if clamp:
                        idx = jnp.minimum(idx, per_w * 4 - 1)
                    cs.append(plsc.load_gather(cont_v, [idx]))
                c0, c1, c2v, c3v = cs
                cc2 = cent2 - c2v
                cc3 = cent3 - c3v
                s0 = alpha * c0 + beta
                s1 = alpha * c1 + beta
                s2 = alpha * cc2 + beta
                s3 = alpha * cc3 + beta
                s4 = plsc.load_gather(sd_v, [d0c[pl.ds(gb, L)]])
                s5 = plsc.load_gather(sd_v, [d1c[pl.ds(gb, L)]])
                s6 = plsc.load_gather(sd_v, [d2c[pl.ds(gb, L)]])
                m = jnp.maximum(
                    jnp.maximum(jnp.maximum(s0, s1), jnp.maximum(s2, s3)),
                    jnp.maximum(jnp.maximum(s4, s5), s6))
                e0 = jnp.exp(s0 - m)
                e1 = jnp.exp(s1 - m)
                e2 = jnp.exp(s2 - m)
                e3 = jnp.exp(s3 - m)
                e4 = jnp.exp(s4 - m)
                e5 = jnp.exp(s5 - m)
                e6 = jnp.exp(s6 - m)
                rr = 1.0 / (((e0 + e1) + (e2 + e3)) + ((e4 + e5) + e6))
                p0 = e0 * rr
                p1 = e1 * rr
                p2 = e2 * rr
                p3 = e3 * rr
                pv0[pl.ds(gb, L)] = e4 * rr
                pv1[pl.ds(gb, L)] = e5 * rr
                pv2[pl.ds(gb, L)] = e6 * rr
                pvw[pl.ds(gb, L)] = p0 * c0 + p1 * c1 + p2 * cc2 + p3 * cc3
                pvs[pl.ds(gb, L)] = (p0 + p1) + (p2 + p3)

            def comb_grp(gb, jn):
                p4g = pv0[pl.ds(gb, L)]
                p5g = pv1[pl.ds(gb, L)]
                p6g = pv2[pl.ds(gb, L)]
                wg = pvw[pl.ds(gb, L)]
                psg = pvs[pl.ds(gb, L)]
                for j in range(jn):
                    i = gb + j
                    p4 = p4g[j]
                    p5 = p5g[j]
                    p6 = p6g[j]
                    w = wg[j]
                    ps = psg[j]
                    for q in range(4):
                        sl = pl.ds(q * L, L)
                        r1[i, sl] = (r0[i, sl] * p4 + r1[i, sl] * p5
                                     + r2[i, sl] * p6
                                     + uvec[q] * w + vvec[q] * ps)

            def chunk(ci, carry):
                o = ci * n
                b3 = o * 3
                b4 = o * 4
                cv = cont_v[pl.ds(b4, L)]
                cent2 = cv[2]
                cent3 = cv[3]

                def dg(g, c2):
                    deint_grp(g * L, b3, False)
                    return c2
                lax.fori_loop(0, n_grp - 1, dg, 0)
                deint_grp((n_grp - 1) * L, b3, True)

                cps = [pltpu.async_copy(
                    t2_hbm.at[dca[a].at[pl.ds(0, n)]], (r0, r1, r2)[a], sem)
                    for a in range(3)]

                def sg(g, c2):
                    soft_grp(g * L, b4, cent2, cent3, False)
                    return c2
                lax.fori_loop(0, n_grp - 1, sg, 0)
                soft_grp((n_grp - 1) * L, b4, cent2, cent3, True)

                for cp in cps:
                    cp.wait()

                def cg(g, c2):
                    comb_grp(g * L, L)
                    return c2
                lax.fori_loop(0, n_grp - 1, cg, 0)
                comb_grp((n_grp - 1) * L, tail)

                pltpu.sync_copy(r1, out_hbm.at[pl.ds(base_w + o, n)])
                return carry

            lax.fori_loop(0, ROWS_PER_W, chunk, 0)

        phase(N_TILE, PW_TILE, cont_t_hbm, disc_t_hbm, sd_t_hbm, par_t_hbm,
              t2_t_hbm, out_t_hbm, r0t, r1t, r2t)
        phase(N_ENT, PW_ENT, cont_e_hbm, disc_e_hbm, sd_e_hbm, par_e_hbm,
              t2_e_hbm, out_e_hbm, r0e, r1e, r2e)

    return k(cont_t, disc_t, cont_e, disc_e,
             sd_t, t2_t, par_t, sd_e, t2_e, par_e)


def kernel(tile_cont, tile_disc, ent_cont, ent_disc, tile_table, ent_table,
           tile_wc, tile_bc, tile_wa, tile_wf,
           ent_wc, ent_bc, ent_wa, ent_wf):
    zpad = jnp.zeros((EMBED, 128 - EMBED - 1), F32)
    wet = jnp.concatenate([tile_wf, tile_wa[:, None], zpad], axis=1)
    wee = jnp.concatenate([ent_wf, ent_wa[:, None], zpad], axis=1)
    wcbt = jnp.stack([tile_wc, tile_bc])
    wcbe = jnp.stack([ent_wc, ent_bc])

    t2xt, t2xe, mt, me = _tc_pre(tile_table, ent_table, wet, wee, wcbt, wcbe)

    out_t, out_e = _sc_fused(
        tile_cont.reshape(P_TILE * 4),
        tile_disc.astype(jnp.int32).reshape(P_TILE * 3),
        ent_cont.reshape(P_ENT * 4),
        ent_disc.astype(jnp.int32).reshape(P_ENT * 3),
        t2xt[:, EMBED], t2xt[:, :EMBED], mt.reshape(256),
        t2xe[:, EMBED], t2xe[:, :EMBED], me.reshape(256))

    return (out_t.reshape(B, N_TILE, EMBED), out_e.reshape(B, N_ENT, EMBED))


# speedup vs baseline: 1.7793x; 1.7713x over previous
import functools

import jax
import jax.numpy as jnp
from jax import lax
from jax.experimental import pallas as pl
from jax.experimental.pallas import tpu as pltpu
from jax.experimental.pallas import tpu_sc as plsc

F32 = jnp.float32

B = 1024
N_TILE = 225
N_ENT = 100
VOCAB = 4096
EMBED = 64

NC = 2
NS = 16
NW = NC * NS
L = 16


def _tc_pre_body(tt, te, wet, wee, wcbt, wcbe, ct, ce,
                 t2t, t2e, mt, me, cct, cce):
    hi = lax.Precision.HIGHEST
    t2t[...] = jnp.dot(tt[...], wet[...], precision=hi)
    t2e[...] = jnp.dot(te[...], wee[...], precision=hi)
    mt[...] = jnp.dot(wcbt[...], wet[...], precision=hi)
    me[...] = jnp.dot(wcbe[...], wee[...], precision=hi)
    for cref, oref in ((ct, cct), (ce, cce)):
        c = cref[...]
        a = lax.broadcasted_iota(jnp.int32, c.shape, 0)
        oref[...] = jnp.where(a >= 2, c[:, :, 0:1] - c, c)


def _tc_pre(tt, te, wet, wee, wcbt, wcbe, ct, ce):
    out_shape = (
        jax.ShapeDtypeStruct((VOCAB, 128), F32),
        jax.ShapeDtypeStruct((VOCAB, 128), F32),
        jax.ShapeDtypeStruct((2, 128), F32),
        jax.ShapeDtypeStruct((2, 128), F32),
        jax.ShapeDtypeStruct((4, B, N_TILE), F32),
        jax.ShapeDtypeStruct((4, B, N_ENT), F32),
    )
    return pl.pallas_call(_tc_pre_body, out_shape=out_shape)(
        tt, te, wet, wee, wcbt, wcbe, ct, ce)


def _sc_branch(P, C, contc, disc, sd, t2, u, v0, ab):
    per_w = P // NW
    n_chunks = per_w // C
    n_grp = C // L
    assert per_w * NW == P and n_chunks * C == per_w and n_grp * L == C

    mesh = plsc.VectorSubcoreMesh(core_axis_name="c", subcore_axis_name="s")

    @functools.partial(
        pl.kernel, mesh=mesh,
        out_type=jax.ShapeDtypeStruct((P, EMBED), F32),
        compiler_params=pltpu.CompilerParams(
            needs_layout_passes=False, use_tc_tiling_on_sc=False),
        scratch_types=(
            [pltpu.VMEM((VOCAB,), F32)]
            + [pltpu.VMEM((EMBED,), F32)] * 2
            + [pltpu.VMEM((L,), F32)]
            + [pltpu.VMEM((per_w,), F32)] * 4
            + [pltpu.VMEM((per_w,), jnp.int32)] * 3
            + [pltpu.VMEM((C, EMBED), F32)] * 3
            + [pltpu.VMEM((C,), F32)] * 5
            + [pltpu.SemaphoreType.DMA]
        ),
    )
    def k(cont_hbm, disc_hbm, sd_hbm, t2_hbm, u_hbm, v0_hbm, ab_hbm, out_hbm,
          sd_v, u_v, v0_v, ab_v, c0_v, c1_v, c2_v, c3_v, d0_v, d1_v, d2_v,
          r0, r1, r2, pv0, pv1, pv2, pvw, pvs, sem):
        wid = lax.axis_index("s") * NC + lax.axis_index("c")
        base_w = wid * per_w
        pltpu.sync_copy(sd_hbm, sd_v)
        pltpu.sync_copy(u_hbm, u_v)
        pltpu.sync_copy(v0_hbm, v0_v)
        pltpu.sync_copy(ab_hbm, ab_v)
        cont_vs = (c0_v, c1_v, c2_v, c3_v)
        disc_vs = (d0_v, d1_v, d2_v)
        for a in range(4):
            pltpu.sync_copy(cont_hbm.at[pl.ds(a * P + base_w, per_w)],
                            cont_vs[a])
        for a in range(3):
            pltpu.sync_copy(disc_hbm.at[pl.ds(a * P + base_w, per_w)],
                            disc_vs[a])
        abv = ab_v[...]
        alpha = abv[0]
        beta = abv[1]
        uvec = [u_v[pl.ds(kk * L, L)] for kk in range(EMBED // L)]
        vvec = [v0_v[pl.ds(kk * L, L)] for kk in range(EMBED // L)]

        def chunk(ci, carry):
            o = ci * C
            cp0 = pltpu.async_copy(t2_hbm.at[d0_v.at[pl.ds(o, C)]], r0, sem)
            cp1 = pltpu.async_copy(t2_hbm.at[d1_v.at[pl.ds(o, C)]], r1, sem)
            cp2 = pltpu.async_copy(t2_hbm.at[d2_v.at[pl.ds(o, C)]], r2, sem)

            def grp(gi, c2):
                og = o + gi * L
                gl = gi * L
                c0 = c0_v[pl.ds(og, L)]
                c1 = c1_v[pl.ds(og, L)]
                cc2 = c2_v[pl.ds(og, L)]
                c3 = c3_v[pl.ds(og, L)]
                s0 = alpha * c0 + beta
                s1 = alpha * c1 + beta
                s2 = alpha * cc2 + beta
                s3 = alpha * c3 + beta
                s4 = plsc.load_gather(sd_v, [d0_v[pl.ds(og, L)]])
                s5 = plsc.load_gather(sd_v, [d1_v[pl.ds(og, L)]])
                s6 = plsc.load_gather(sd_v, [d2_v[pl.ds(og, L)]])
                m = jnp.maximum(
                    jnp.maximum(jnp.maximum(s0, s1), jnp.maximum(s2, s3)),
                    jnp.maximum(jnp.maximum(s4, s5), s6))
                e0 = jnp.exp(s0 - m)
                e1 = jnp.exp(s1 - m)
                e2 = jnp.exp(s2 - m)
                e3 = jnp.exp(s3 - m)
                e4 = jnp.exp(s4 - m)
                e5 = jnp.exp(s5 - m)
                e6 = jnp.exp(s6 - m)
                r = 1.0 / (((e0 + e1) + (e2 + e3)) + ((e4 + e5) + e6))
                p0 = e0 * r
                p1 = e1 * r
                p2 = e2 * r
                p3 = e3 * r
                pv0[pl.ds(gl, L)] = e4 * r
                pv1[pl.ds(gl, L)] = e5 * r
                pv2[pl.ds(gl, L)] = e6 * r
                pvw[pl.ds(gl, L)] = p0 * c0 + p1 * c1 + p2 * cc2 + p3 * c3
                pvs[pl.ds(gl, L)] = (p0 + p1) + (p2 + p3)
                return c2

            lax.fori_loop(0, n_grp, grp, 0, unroll=True)
            cp0.wait()
            cp1.wait()
            cp2.wait()

            def posg(gi, c2):
                gl = gi * L
                p4g = pv0[pl.ds(gl, L)]
                p5g = pv1[pl.ds(gl, L)]
                p6g = pv2[pl.ds(gl, L)]
                wg = pvw[pl.ds(gl, L)]
                psg = pvs[pl.ds(gl, L)]
                for j in range(L):
                    i = gl + j
                    p4 = p4g[j]
                    p5 = p5g[j]
                    p6 = p6g[j]
                    w = wg[j]
                    ps = psg[j]
                    for kk in range(EMBED // L):
                        sl = pl.ds(kk * L, L)
                        r0[i, sl] = (r0[i, sl] * p4 + r1[i, sl] * p5
                                     + r2[i, sl] * p6
                                     + uvec[kk] * w + vvec[kk] * ps)
                return c2

            lax.fori_loop(0, n_grp, posg, 0)
            pltpu.sync_copy(r0, out_hbm.at[pl.ds(base_w + o, C)])
            return carry

        lax.fori_loop(0, n_chunks, chunk, 0)

    return k(contc, disc, sd, t2, u, v0, ab)


def kernel(tile_cont, tile_disc, ent_cont, ent_disc, tile_table, ent_table,
           tile_wc, tile_bc, tile_wa, tile_wf,
           ent_wc, ent_bc, ent_wa, ent_wf):
    zpad = jnp.zeros((EMBED, 63), F32)
    wet = jnp.concatenate([tile_wf, tile_wa[:, None], zpad], axis=1)
    wee = jnp.concatenate([ent_wf, ent_wa[:, None], zpad], axis=1)
    wcbt = jnp.stack([tile_wc, tile_bc])
    wcbe = jnp.stack([ent_wc, ent_bc])
    ct = jnp.transpose(tile_cont, (2, 0, 1))
    ce = jnp.transpose(ent_cont, (2, 0, 1))

    t2t, t2e, mt, me, cct, cce = _tc_pre(
        tile_table, ent_table, wet, wee, wcbt, wcbe, ct, ce)

    outs = []
    for (t2x, mx, ccx, disc, n, c_chunk) in (
            (t2t, mt, cct, tile_disc, N_TILE, 96),
            (t2e, me, cce, ent_disc, N_ENT, 128)):
        p = B * n
        t2 = t2x[:, :EMBED]
        sd = t2x[:, EMBED]
        u = mx[0, :EMBED]
        v0 = mx[1, :EMBED]
        ab = jnp.concatenate([mx[:, EMBED], jnp.zeros((L - 2,), F32)])
        contc = ccx.reshape(4 * p)
        disc_t = jnp.transpose(disc, (2, 0, 1)).reshape(3 * p).astype(jnp.int32)
        out = _sc_branch(p, c_chunk, contc, disc_t, sd, t2, u, v0, ab)
        outs.append(out.reshape(B, n, EMBED))
    return (outs[0], outs[1])
